# Initial kernel scaffold; baseline (speedup 1.0000x reference)
import functools

import jax
import jax.numpy as jnp
from jax import lax
from jax.experimental import pallas as pl
from jax.experimental.pallas import tpu as pltpu
from jax.experimental.pallas import tpu_sc as plsc

N = 10000
D = 128
E = 106667
NUM_REL = 3
NUM_LAYERS = 2

NC = 2
NS = 16
LANES = 16
DC = D // NC
QN = DC // LANES

K = 128
EPT_RAW = -(-E // NS)
NCHUNK = -(-EPT_RAW // K)
EPT = NCHUNK * K
E_PAD = EPT * NS

RN = N // NS
ZR = 125
NU = RN // ZR


def _forward(xin, rows, cols, wts):
  mesh = plsc.VectorSubcoreMesh(core_axis_name="c", subcore_axis_name="s")

  @functools.partial(
      pl.kernel,
      out_type=jax.ShapeDtypeStruct((NC, N, DC), jnp.float32),
      mesh=mesh,
      scratch_types=[
          pltpu.VMEM_SHARED((N, DC), jnp.float32),
          pltpu.VMEM_SHARED((N, DC), jnp.float32),
          pltpu.VMEM((NCHUNK, K), jnp.int32),
          pltpu.VMEM((NCHUNK, K), jnp.int32),
          pltpu.VMEM((NCHUNK, K), jnp.float32),
          pltpu.VMEM((K, DC), jnp.float32),
          pltpu.VMEM((RN, DC), jnp.float32),
          pltpu.VMEM((ZR, DC), jnp.float32),
          pltpu.VMEM((ZR, DC), jnp.float32),
          pltpu.SemaphoreType.DMA,
      ],
  )
  def body(xin_hbm, rows_hbm, cols_hbm, wts_hbm, out_hbm,
           x_sp, acc_sp, rows_v, cols_v, w_v, gbuf, laysum, tmp, zbuf, sem):
    c = lax.axis_index("c")
    s = lax.axis_index("s")
    base = s * RN

    zero16 = jnp.zeros((LANES,), jnp.float32)

    def zero_row(i, _):
      for q in range(QN):
        zbuf[i, pl.ds(q * LANES, LANES)] = zero16
      return 0
    lax.fori_loop(0, ZR, zero_row, 0)

    for u in range(NU):
      off = base + u * ZR
      pltpu.sync_copy(xin_hbm.at[c, pl.ds(off, ZR)], tmp)
      pltpu.sync_copy(tmp, x_sp.at[pl.ds(off, ZR)])
      pltpu.sync_copy(zbuf, acc_sp.at[pl.ds(off, ZR)])
    plsc.subcore_barrier()

    for layer in range(NUM_LAYERS):
      for r in range(NUM_REL):
        pltpu.sync_copy(rows_hbm.at[r, s], rows_v)
        pltpu.sync_copy(cols_hbm.at[r, s], cols_v)
        pltpu.sync_copy(wts_hbm.at[r, s], w_v)

        def chunk(j, _):
          pltpu.async_copy(x_sp.at[cols_v.at[j]], gbuf, sem).wait()

          def scale16(b, _):
            w16 = w_v[j, pl.ds(b * LANES, LANES)]
            for e in range(LANES):
              we = jnp.take(w16, jnp.full((LANES,), e, jnp.int32),
                            mode="promise_in_bounds")
              row = b * LANES + e
              for q in range(QN):
                sl = pl.ds(q * LANES, LANES)
                gbuf[row, sl] = gbuf[row, sl] * we
            return 0
          lax.fori_loop(0, K // LANES, scale16, 0)

          pltpu.sync_copy(gbuf, acc_sp.at[rows_v.at[j]], add=True)
          return 0
        lax.fori_loop(0, NCHUNK, chunk, 0)
        plsc.subcore_barrier()

        for u in range(NU):
          off = base + u * ZR
          pltpu.sync_copy(acc_sp.at[pl.ds(off, ZR)], tmp)
          pltpu.sync_copy(zbuf, acc_sp.at[pl.ds(off, ZR)])

          def leaky(i, _):
            for q in range(QN):
              sl = pl.ds(q * LANES, LANES)
              v = tmp[i, sl]
              lv = jnp.maximum(v, v * 0.01)
              if r == 0:
                laysum[u * ZR + i, sl] = lv
              else:
                laysum[u * ZR + i, sl] = laysum[u * ZR + i, sl] + lv
            return 0
          lax.fori_loop(0, ZR, leaky, 0)
        plsc.subcore_barrier()

      if layer < NUM_LAYERS - 1:
        pltpu.sync_copy(laysum, x_sp.at[pl.ds(base, RN)])
        plsc.subcore_barrier()
      else:
        pltpu.sync_copy(laysum, out_hbm.at[c, pl.ds(base, RN)])

  return body(xin, rows, cols, wts)


def _prep_edges(edge_index, edge_weight):
  rows = edge_index[0]
  cols = edge_index[1]
  pad = E_PAD - E
  rows = jnp.concatenate([rows, jnp.zeros((pad,), rows.dtype)])
  cols = jnp.concatenate([cols, jnp.zeros((pad,), cols.dtype)])
  w = jnp.concatenate([edge_weight, jnp.zeros((pad,), edge_weight.dtype)])
  return (rows.reshape(NS, NCHUNK, K), cols.reshape(NS, NCHUNK, K),
          w.reshape(NS, NCHUNK, K))


@jax.jit
def kernel(init_embs, edge_index_r0, edge_weight_r0, edge_index_r1,
           edge_weight_r1, edge_index_r2, edge_weight_r2):
  xin = jnp.stack([init_embs[:, :DC], init_embs[:, DC:]])
  r0 = _prep_edges(edge_index_r0, edge_weight_r0)
  r1 = _prep_edges(edge_index_r1, edge_weight_r1)
  r2 = _prep_edges(edge_index_r2, edge_weight_r2)
  rows = jnp.stack([r0[0], r1[0], r2[0]])
  cols = jnp.stack([r0[1], r1[1], r2[1]])
  wts = jnp.stack([r0[2], r1[2], r2[2]])
  out = _forward(xin, rows, cols, wts)
  return jnp.concatenate([out[0], out[1]], axis=1)


# speedup vs baseline: 5.2458x; 5.2458x over previous
import functools

import jax
import jax.numpy as jnp
from jax import lax
from jax.experimental import pallas as pl
from jax.experimental.pallas import tpu as pltpu
from jax.experimental.pallas import tpu_sc as plsc

N = 10000
D = 128
E = 106667
NUM_REL = 3
NUM_LAYERS = 2

NC = 2
NS = 16
LANES = 16
DC = D // NC
QN = DC // LANES

K = 128
EPT_RAW = -(-E // NS)
NCHUNK = -(-EPT_RAW // K)
EPT = NCHUNK * K
E_PAD = EPT * NS

N_PAD = 10240
RN = N_PAD // NS
ZR = 32
NU = RN // ZR


def _forward(xin, rows, cols, wts):
  mesh = plsc.VectorSubcoreMesh(core_axis_name="c", subcore_axis_name="s")

  @functools.partial(
      pl.kernel,
      out_type=jax.ShapeDtypeStruct((NC * N_PAD, DC), jnp.float32),
      mesh=mesh,
      scratch_types=[
          pltpu.VMEM_SHARED((N_PAD, DC), jnp.float32),
          pltpu.VMEM_SHARED((N_PAD, DC), jnp.float32),
          pltpu.VMEM((NCHUNK, K), jnp.int32),
          pltpu.VMEM((NCHUNK, K), jnp.int32),
          pltpu.VMEM((NCHUNK, K), jnp.float32),
          pltpu.VMEM((K, DC), jnp.float32),
          pltpu.VMEM((ZR, DC), jnp.float32),
          pltpu.VMEM((ZR, DC), jnp.float32),
          pltpu.SemaphoreType.DMA,
      ],
  )
  def body(xin_hbm, rows_hbm, cols_hbm, wts_hbm, out_hbm,
           x_sp, acc_sp, rows_v, cols_v, w_v, gbuf, tmp, tmp2, sem):
    c = lax.axis_index("c")
    s = lax.axis_index("s")
    base = s * RN

    zero16 = jnp.zeros((LANES,), jnp.float32)
    zsrc = gbuf.at[pl.ds(0, ZR)]

    def zero_gbuf():
      def zero_row(i, _):
        for q in range(QN):
          gbuf[i, pl.ds(q * LANES, LANES)] = zero16
        return 0
      lax.fori_loop(0, ZR, zero_row, 0)

    zero_gbuf()
    for u in range(NU):
      off = base + u * ZR
      pltpu.sync_copy(xin_hbm.at[pl.ds(c * N_PAD + off, ZR)], tmp)
      pltpu.sync_copy(tmp, x_sp.at[pl.ds(off, ZR)])
      pltpu.sync_copy(zsrc, acc_sp.at[pl.ds(off, ZR)])
    plsc.subcore_barrier()

    for layer in range(NUM_LAYERS):
      for r in range(NUM_REL):
        pltpu.sync_copy(rows_hbm.at[r, s], rows_v)
        pltpu.sync_copy(cols_hbm.at[r, s], cols_v)
        pltpu.sync_copy(wts_hbm.at[r, s], w_v)

        def chunk(j, _):
          pltpu.async_copy(x_sp.at[cols_v.at[j]], gbuf, sem).wait()

          def scale16(b, _):
            w16v = w_v[j, pl.ds(b * LANES, LANES)]
            for e in range(LANES):
              row = b * LANES + e
              we = jnp.full((LANES,), w16v[e], jnp.float32)
              for q in range(QN):
                sl = pl.ds(q * LANES, LANES)
                gbuf[row, sl] = gbuf[row, sl] * we
            return 0
          lax.fori_loop(0, K // LANES, scale16, 0)

          pltpu.sync_copy(gbuf, acc_sp.at[rows_v.at[j]], add=True)
          return 0
        lax.fori_loop(0, NCHUNK, chunk, 0)
        plsc.subcore_barrier()

        zero_gbuf()
        for u in range(NU):
          off = base + u * ZR
          pltpu.sync_copy(acc_sp.at[pl.ds(off, ZR)], tmp)
          pltpu.sync_copy(zsrc, acc_sp.at[pl.ds(off, ZR)])
          if r > 0:
            pltpu.sync_copy(out_hbm.at[pl.ds(c * N_PAD + off, ZR)], tmp2)

          def leaky(i, _):
            for q in range(QN):
              sl = pl.ds(q * LANES, LANES)
              v = tmp[i, sl]
              lv = jnp.maximum(v, v * 0.01)
              if r == 0:
                tmp[i, sl] = lv
              else:
                tmp2[i, sl] = tmp2[i, sl] + lv
            return 0
          lax.fori_loop(0, ZR, leaky, 0)

          src = tmp if r == 0 else tmp2
          pltpu.sync_copy(src, out_hbm.at[pl.ds(c * N_PAD + off, ZR)])
        plsc.subcore_barrier()

      if layer < NUM_LAYERS - 1:
        for u in range(NU):
          off = base + u * ZR
          pltpu.sync_copy(out_hbm.at[pl.ds(c * N_PAD + off, ZR)], tmp)
          pltpu.sync_copy(tmp, x_sp.at[pl.ds(off, ZR)])
        plsc.subcore_barrier()

  return body(xin, rows, cols, wts)


def _prep_edges(edge_index, edge_weight):
  rows = edge_index[0]
  cols = edge_index[1]
  pad = E_PAD - E
  rows = jnp.concatenate([rows, jnp.zeros((pad,), rows.dtype)])
  cols = jnp.concatenate([cols, jnp.zeros((pad,), cols.dtype)])
  w = jnp.concatenate([edge_weight, jnp.zeros((pad,), edge_weight.dtype)])
  return (rows.reshape(NS, NCHUNK, K), cols.reshape(NS, NCHUNK, K),
          w.reshape(NS, NCHUNK, K))


@jax.jit
def kernel(init_embs, edge_index_r0, edge_weight_r0, edge_index_r1,
           edge_weight_r1, edge_index_r2, edge_weight_r2):
  xpad = jnp.concatenate(
      [init_embs, jnp.zeros((N_PAD - N, D), init_embs.dtype)])
  xin = jnp.concatenate([xpad[:, :DC], xpad[:, DC:]])
  r0 = _prep_edges(edge_index_r0, edge_weight_r0)
  r1 = _prep_edges(edge_index_r1, edge_weight_r1)
  r2 = _prep_edges(edge_index_r2, edge_weight_r2)
  rows = jnp.stack([r0[0], r1[0], r2[0]])
  cols = jnp.stack([r0[1], r1[1], r2[1]])
  wts = jnp.stack([r0[2], r1[2], r2[2]])
  out = _forward(xin, rows, cols, wts)
  return jnp.concatenate([out[:N], out[N_PAD:N_PAD + N]], axis=1)


# speedup vs baseline: 5.7832x; 1.1024x over previous
import functools

import jax
import jax.numpy as jnp
from jax import lax
from jax.experimental import pallas as pl
from jax.experimental.pallas import tpu as pltpu
from jax.experimental.pallas import tpu_sc as plsc

N = 10000
D = 128
E = 106667
NUM_REL = 3
NUM_LAYERS = 2

NC = 2
NS = 16
LANES = 16
DC = D // NC
QN = DC // LANES

K = 64
EPT_RAW = -(-E // NS)
NCHUNK = -(-EPT_RAW // K)
NCHUNK += NCHUNK % 2
EPT = NCHUNK * K
E_PAD = EPT * NS

N_PAD = 10240
RN = N_PAD // NS
ZR = 32
NU = RN // ZR


def _forward(xin, rows, cols, wts):
  mesh = plsc.VectorSubcoreMesh(core_axis_name="c", subcore_axis_name="s")

  @functools.partial(
      pl.kernel,
      out_type=jax.ShapeDtypeStruct((NC * N_PAD, DC), jnp.float32),
      mesh=mesh,
      scratch_types=[
          pltpu.VMEM_SHARED((N_PAD, DC), jnp.float32),
          pltpu.VMEM_SHARED((N_PAD, DC), jnp.float32),
          pltpu.VMEM((NCHUNK, K), jnp.int32),
          pltpu.VMEM((NCHUNK // 2, 2 * K), jnp.int32),
          pltpu.VMEM((NCHUNK // 2, 2 * K), jnp.float32),
          pltpu.VMEM((K, DC), jnp.float32),
          pltpu.VMEM((K, DC), jnp.float32),
          pltpu.SemaphoreType.DMA,
          pltpu.SemaphoreType.DMA,
      ],
  )
  def body(xin_hbm, rows_hbm, cols_hbm, wts_hbm, out_hbm,
           x_sp, acc_sp, rows_v, cols_v, w_v, g0, g1, gsem, ssem):
    c = lax.axis_index("c")
    s = lax.axis_index("s")
    base = s * RN

    zero16 = jnp.zeros((LANES,), jnp.float32)
    zsrc = g0.at[pl.ds(0, ZR)]
    tmp = g0.at[pl.ds(ZR, ZR)]
    tmp2 = g1.at[pl.ds(0, ZR)]

    def zero_gbuf():
      def zero_row(i, _):
        for q in range(QN):
          g0[i, pl.ds(q * LANES, LANES)] = zero16
        return 0
      lax.fori_loop(0, ZR, zero_row, 0)

    zero_gbuf()

    def init_u(u, _):
      off = pl.multiple_of(base + u * ZR, ZR)
      pltpu.sync_copy(xin_hbm.at[pl.ds(c * N_PAD + off, ZR)], tmp)
      pltpu.sync_copy(tmp, x_sp.at[pl.ds(off, ZR)])
      pltpu.sync_copy(zsrc, acc_sp.at[pl.ds(off, ZR)])
      return 0
    lax.fori_loop(0, NU, init_u, 0)
    plsc.subcore_barrier()

    for layer in range(NUM_LAYERS):
      for r in range(NUM_REL):
        pltpu.sync_copy(rows_hbm.at[r, s], rows_v)
        pltpu.sync_copy(cols_hbm.at[r, s], cols_v)
        pltpu.sync_copy(wts_hbm.at[r, s], w_v)

        def scale(j2, par, buf):
          def scale16(b, _):
            w16v = w_v[j2, pl.ds(par * K + b * LANES, LANES)]
            for e in range(LANES):
              row = b * LANES + e
              we = jnp.full((LANES,), w16v[e], jnp.float32)
              for q in range(QN):
                sl = pl.ds(q * LANES, LANES)
                buf[row, sl] = buf[row, sl] * we
            return 0
          lax.fori_loop(0, K // LANES, scale16, 0)

        def start_gather(j2, par, buf):
          pltpu.async_copy(
              x_sp.at[cols_v.at[j2, pl.ds(par * K, K)]], buf, gsem)

        def wait_gather(j2, par, buf):
          pltpu.make_async_copy(
              x_sp.at[cols_v.at[j2, pl.ds(par * K, K)]], buf, gsem).wait()

        def start_scatter(j, buf):
          pltpu.async_copy(buf, acc_sp.at[rows_v.at[j]], ssem, add=True)

        def wait_scatter(j, buf):
          pltpu.make_async_copy(buf, acc_sp.at[rows_v.at[j]], ssem).wait()

        start_gather(0, 0, g0)

        def pair(j2, _):
          a = 2 * j2
          wait_gather(j2, 0, g0)

          @pl.when(j2 > 0)
          def _():
            wait_scatter(a - 1, g1)
          start_gather(j2, 1, g1)
          scale(j2, 0, g0)
          start_scatter(a, g0)
          wait_gather(j2, 1, g1)
          wait_scatter(a, g0)

          @pl.when(j2 < NCHUNK // 2 - 1)
          def _():
            start_gather(j2 + 1, 0, g0)
          scale(j2, 1, g1)
          start_scatter(a + 1, g1)
          return 0
        lax.fori_loop(0, NCHUNK // 2, pair, 0)
        wait_scatter(NCHUNK - 1, g1)
        plsc.subcore_barrier()

        zero_gbuf()

        def leaky_u(u, _):
          off = pl.multiple_of(base + u * ZR, ZR)
          pltpu.sync_copy(acc_sp.at[pl.ds(off, ZR)], tmp)
          pltpu.sync_copy(zsrc, acc_sp.at[pl.ds(off, ZR)])
          if r > 0:
            pltpu.sync_copy(out_hbm.at[pl.ds(c * N_PAD + off, ZR)], tmp2)

          def leaky(i, _):
            for q in range(QN):
              sl = pl.ds(q * LANES, LANES)
              v = g0[ZR + i, sl]
              lv = jnp.maximum(v, v * 0.01)
              if r == 0:
                g0[ZR + i, sl] = lv
              else:
                g1[i, sl] = g1[i, sl] + lv
            return 0
          lax.fori_loop(0, ZR, leaky, 0)

          src = tmp if r == 0 else tmp2
          pltpu.sync_copy(src, out_hbm.at[pl.ds(c * N_PAD + off, ZR)])
          return 0
        lax.fori_loop(0, NU, leaky_u, 0)
        plsc.subcore_barrier()

      if layer < NUM_LAYERS - 1:
        def readback_u(u, _):
          off = pl.multiple_of(base + u * ZR, ZR)
          pltpu.sync_copy(out_hbm.at[pl.ds(c * N_PAD + off, ZR)], tmp)
          pltpu.sync_copy(tmp, x_sp.at[pl.ds(off, ZR)])
          return 0
        lax.fori_loop(0, NU, readback_u, 0)
        plsc.subcore_barrier()

  return body(xin, rows, cols, wts)


def _prep_edges(edge_index, edge_weight):
  rows = edge_index[0]
  cols = edge_index[1]
  pad = E_PAD - E
  rows = jnp.concatenate([rows, jnp.zeros((pad,), rows.dtype)])
  cols = jnp.concatenate([cols, jnp.zeros((pad,), cols.dtype)])
  w = jnp.concatenate([edge_weight, jnp.zeros((pad,), edge_weight.dtype)])
  return (rows.reshape(NS, NCHUNK, K), cols.reshape(NS, NCHUNK // 2, 2 * K),
          w.reshape(NS, NCHUNK // 2, 2 * K))


@jax.jit
def kernel(init_embs, edge_index_r0, edge_weight_r0, edge_index_r1,
           edge_weight_r1, edge_index_r2, edge_weight_r2):
  xpad = jnp.concatenate(
      [init_embs, jnp.zeros((N_PAD - N, D), init_embs.dtype)])
  xin = jnp.concatenate([xpad[:, :DC], xpad[:, DC:]])
  r0 = _prep_edges(edge_index_r0, edge_weight_r0)
  r1 = _prep_edges(edge_index_r1, edge_weight_r1)
  r2 = _prep_edges(edge_index_r2, edge_weight_r2)
  rows = jnp.stack([r0[0], r1[0], r2[0]])
  cols = jnp.stack([r0[1], r1[1], r2[1]])
  wts = jnp.stack([r0[2], r1[2], r2[2]])
  out = _forward(xin, rows, cols, wts)
  return jnp.concatenate([out[:N], out[N_PAD:N_PAD + N]], axis=1)


# speedup vs baseline: 6.7147x; 1.1611x over previous
import functools

import jax
import jax.numpy as jnp
from jax import lax
from jax.experimental import pallas as pl
from jax.experimental.pallas import tpu as pltpu
from jax.experimental.pallas import tpu_sc as plsc

N = 10000
D = 128
E = 106667
NUM_REL = 3
NUM_LAYERS = 2

NC = 2
NS = 16
LANES = 16
DC = D // NC
QN = DC // LANES

K = 64
EPT_RAW = -(-E // NS)
NCHUNK = -(-EPT_RAW // K)
NCHUNK += NCHUNK % 2
EPT = NCHUNK * K
E_PAD = EPT * NS

N_PAD = 10240
RN = N_PAD // NS
ZR = 32
NU = RN // ZR


def _forward(xin, rows, cols, wts):
  mesh = plsc.VectorSubcoreMesh(core_axis_name="c", subcore_axis_name="s")

  @functools.partial(
      pl.kernel,
      out_type=jax.ShapeDtypeStruct((NC * N_PAD, DC), jnp.float32),
      mesh=mesh,
      scratch_types=[
          pltpu.VMEM_SHARED((N_PAD, DC), jnp.float32),
          pltpu.VMEM_SHARED((N_PAD, DC), jnp.float32),
          pltpu.VMEM((NCHUNK, K), jnp.int32),
          pltpu.VMEM((NCHUNK // 2, 2 * K), jnp.int32),
          pltpu.VMEM((NCHUNK // 2, 2 * K), jnp.float32),
          pltpu.VMEM((K, DC), jnp.float32),
          pltpu.VMEM((K, DC), jnp.float32),
          pltpu.VMEM((ZR, DC), jnp.float32),
          pltpu.SemaphoreType.DMA,
          pltpu.SemaphoreType.DMA,
          pltpu.SemaphoreType.DMA,
          pltpu.SemaphoreType.DMA,
          pltpu.SemaphoreType.DMA,
      ],
  )
  def body(xin_hbm, rows_hbm, cols_hbm, wts_hbm, out_hbm,
           x_sp, acc_sp, rows_v, cols_v, w_v, g0, g1, zb,
           gsem, ssem, asem, osem, vsem):
    c = lax.axis_index("c")
    s = lax.axis_index("s")
    base = s * RN

    zero16 = jnp.zeros((LANES,), jnp.float32)
    ta = g0.at[pl.ds(0, ZR)]
    tb = g0.at[pl.ds(ZR, ZR)]
    oa = g1.at[pl.ds(0, ZR)]
    ob = g1.at[pl.ds(ZR, ZR)]

    def zero_zb(i, _):
      for q in range(QN):
        zb[i, pl.ds(q * LANES, LANES)] = zero16
      return 0
    lax.fori_loop(0, ZR, zero_zb, 0)

    def off_of(u):
      return pl.multiple_of(base + u * ZR, ZR)

    def init_u(u, _):
      off = off_of(u)
      pltpu.sync_copy(xin_hbm.at[pl.ds(c * N_PAD + off, ZR)], ta)
      pltpu.sync_copy(ta, x_sp.at[pl.ds(off, ZR)])
      pltpu.sync_copy(zb, acc_sp.at[pl.ds(off, ZR)])
      return 0
    lax.fori_loop(0, NU, init_u, 0)
    plsc.subcore_barrier()

    for layer in range(NUM_LAYERS):
      for r in range(NUM_REL):
        pltpu.sync_copy(rows_hbm.at[r, s], rows_v)
        pltpu.sync_copy(cols_hbm.at[r, s], cols_v)
        pltpu.sync_copy(wts_hbm.at[r, s], w_v)

        def scale(j2, par, buf):
          def scale16(b, _):
            w16v = w_v[j2, pl.ds(par * K + b * LANES, LANES)]
            for e in range(LANES):
              row = b * LANES + e
              we = jnp.full((LANES,), w16v[e], jnp.float32)
              for q in range(QN):
                sl = pl.ds(q * LANES, LANES)
                buf[row, sl] = buf[row, sl] * we
            return 0
          lax.fori_loop(0, K // LANES, scale16, 0)

        def start_gather(j2, par, buf):
          pltpu.async_copy(
              x_sp.at[cols_v.at[j2, pl.ds(par * K, K)]], buf, gsem)

        def wait_gather(j2, par, buf):
          pltpu.make_async_copy(
              x_sp.at[cols_v.at[j2, pl.ds(par * K, K)]], buf, gsem).wait()

        def start_scatter(j, buf):
          pltpu.async_copy(buf, acc_sp.at[rows_v.at[j]], ssem, add=True)

        def wait_scatter(j, buf):
          pltpu.make_async_copy(buf, acc_sp.at[rows_v.at[j]], ssem).wait()

        start_gather(0, 0, g0)

        def pair(j2, _):
          a = 2 * j2
          wait_gather(j2, 0, g0)

          @pl.when(j2 > 0)
          def _():
            wait_scatter(a - 1, g1)
          start_gather(j2, 1, g1)
          scale(j2, 0, g0)
          start_scatter(a, g0)
          wait_gather(j2, 1, g1)
          wait_scatter(a, g0)

          @pl.when(j2 < NCHUNK // 2 - 1)
          def _():
            start_gather(j2 + 1, 0, g0)
          scale(j2, 1, g1)
          start_scatter(a + 1, g1)
          return 0
        lax.fori_loop(0, NCHUNK // 2, pair, 0)
        wait_scatter(NCHUNK - 1, g1)
        plsc.subcore_barrier()

        def start_aread(u, buf):
          pltpu.async_copy(acc_sp.at[pl.ds(off_of(u), ZR)], buf, asem)

        def wait_aread(u, buf):
          pltpu.make_async_copy(
              acc_sp.at[pl.ds(off_of(u), ZR)], buf, asem).wait()

        def zero_acc(u):
          pltpu.sync_copy(zb, acc_sp.at[pl.ds(off_of(u), ZR)])

        def start_oread(u, buf):
          pltpu.async_copy(
              out_hbm.at[pl.ds(c * N_PAD + off_of(u), ZR)], buf, osem)

        def wait_oread(u, buf):
          pltpu.make_async_copy(
              out_hbm.at[pl.ds(c * N_PAD + off_of(u), ZR)], buf, osem).wait()

        def start_write(u, buf):
          pltpu.async_copy(
              buf, out_hbm.at[pl.ds(c * N_PAD + off_of(u), ZR)], vsem)

        def wait_write(u, buf):
          pltpu.make_async_copy(
              buf, out_hbm.at[pl.ds(c * N_PAD + off_of(u), ZR)], vsem).wait()

        def compute(zoff):
          def leaky(i, _):
            for q in range(QN):
              sl = pl.ds(q * LANES, LANES)
              v = g0[zoff + i, sl]
              lv = jnp.maximum(v, v * 0.01)
              if r == 0:
                g0[zoff + i, sl] = lv
              else:
                g1[zoff + i, sl] = g1[zoff + i, sl] + lv
            return 0
          lax.fori_loop(0, ZR, leaky, 0)

        wa = ta if r == 0 else oa
        wb = tb if r == 0 else ob

        start_aread(0, ta)
        if r > 0:
          start_oread(0, oa)

        def epair(u2, _):
          ua = 2 * u2
          ub = ua + 1
          wait_aread(ua, ta)
          zero_acc(ua)

          @pl.when(u2 > 0)
          def _():
            wait_write(ub - 2, wb)
          start_aread(ub, tb)
          if r > 0:
            start_oread(ub, ob)
            wait_oread(ua, oa)
          compute(0)
          start_write(ua, wa)
          wait_aread(ub, tb)
          zero_acc(ub)
          wait_write(ua, wa)

          @pl.when(u2 < NU // 2 - 1)
          def _():
            start_aread(ub + 1, ta)
            if r > 0:
              start_oread(ub + 1, oa)
          if r > 0:
            wait_oread(ub, ob)
          compute(ZR)
          start_write(ub, wb)
          return 0
        lax.fori_loop(0, NU // 2, epair, 0)
        wait_write(NU - 1, wb)
        plsc.subcore_barrier()

      if layer < NUM_LAYERS - 1:
        def readback_u(u, _):
          off = off_of(u)
          pltpu.sync_copy(out_hbm.at[pl.ds(c * N_PAD + off, ZR)], ta)
          pltpu.sync_copy(ta, x_sp.at[pl.ds(off, ZR)])
          return 0
        lax.fori_loop(0, NU, readback_u, 0)
        plsc.subcore_barrier()

  return body(xin, rows, cols, wts)


def _prep_edges(edge_index, edge_weight):
  rows = edge_index[0]
  cols = edge_index[1]
  pad = E_PAD - E
  rows = jnp.concatenate([rows, jnp.zeros((pad,), rows.dtype)])
  cols = jnp.concatenate([cols, jnp.zeros((pad,), cols.dtype)])
  w = jnp.concatenate([edge_weight, jnp.zeros((pad,), edge_weight.dtype)])
  return (rows.reshape(NS, NCHUNK, K), cols.reshape(NS, NCHUNK // 2, 2 * K),
          w.reshape(NS, NCHUNK // 2, 2 * K))


@jax.jit
def kernel(init_embs, edge_index_r0, edge_weight_r0, edge_index_r1,
           edge_weight_r1, edge_index_r2, edge_weight_r2):
  xpad = jnp.concatenate(
      [init_embs, jnp.zeros((N_PAD - N, D), init_embs.dtype)])
  xin = jnp.concatenate([xpad[:, :DC], xpad[:, DC:]])
  r0 = _prep_edges(edge_index_r0, edge_weight_r0)
  r1 = _prep_edges(edge_index_r1, edge_weight_r1)
  r2 = _prep_edges(edge_index_r2, edge_weight_r2)
  rows = jnp.stack([r0[0], r1[0], r2[0]])
  cols = jnp.stack([r0[1], r1[1], r2[1]])
  wts = jnp.stack([r0[2], r1[2], r2[2]])
  out = _forward(xin, rows, cols, wts)
  return jnp.concatenate([out[:N], out[N_PAD:N_PAD + N]], axis=1)
